# baseline (speedup 1.0000x reference)
import functools

import jax
import jax.numpy as jnp
from jax import lax
from jax.experimental import pallas as pl
from jax.experimental.pallas import tpu as pltpu
from jax.experimental.pallas import tpu_sc as plsc

N = 10000
E = 320000
KD = 128
MD = 16
NPAD = 10240
EP = 327680
BG = 2 * EP
BE = 512
NBLK = EP // BE
D1 = 640
NC = 2
NS = 16
NW = NC * NS
CH = 128

_f32 = jnp.float32
_i32 = jnp.int32


def _mesh():
    return plsc.VectorSubcoreMesh(
        core_axis_name="c", subcore_axis_name="s", num_cores=NC, num_subcores=NS
    )


@functools.partial(jax.jit, static_argnames=("ncpw", "d"))
def _sc_gather(table, idx2d, *, ncpw, d):
    b = idx2d.shape[0] * CH
    kfire = 4
    v = table.shape[0]
    rpt = v // NS

    def body(table_hbm, idx_hbm, out_hbm, idx_v, rows_v, tab_sh, sems):
        cid = lax.axis_index("c")
        sid = lax.axis_index("s")
        wid = sid * NC + cid
        pltpu.sync_copy(idx_hbm.at[pl.ds(wid * ncpw, ncpw)], idx_v)
        pltpu.sync_copy(
            table_hbm.at[pl.ds(sid * rpt, rpt)], tab_sh.at[pl.ds(sid * rpt, rpt)]
        )
        plsc.subcore_barrier()

        def chunk(i, _):
            c0 = wid * ncpw + i * kfire
            descs = [
                pltpu.async_copy(
                    tab_sh.at[idx_v.at[i * kfire + k]], rows_v.at[k], sems.at[k]
                )
                for k in range(kfire)
            ]
            for k in range(kfire):
                descs[k].wait()
                pltpu.sync_copy(rows_v.at[k], out_hbm.at[pl.ds((c0 + k) * CH, CH)])
            return _

        lax.fori_loop(0, ncpw // kfire, chunk, 0)

    f = pl.kernel(
        body,
        out_type=jax.ShapeDtypeStruct((b, d), table.dtype),
        mesh=_mesh(),
        scratch_types=[
            pltpu.VMEM((ncpw, CH), _i32),
            pltpu.VMEM((kfire, CH, d), table.dtype),
            pltpu.VMEM_SHARED((v, d), table.dtype),
            pltpu.SemaphoreType.DMA((kfire,)),
        ],
        compiler_params=pltpu.CompilerParams(use_tc_tiling_on_sc=False),
    )
    return f(table, idx2d)


@functools.partial(jax.jit, static_argnames=("ncpw", "nr"))
def _sc_scatter_add(m, idx2d, zeros, *, ncpw, nr):
    rpt = nr // NS

    def body(m_hbm, idx_hbm, z_hbm, out_hbm, idx_v, rows_v, acc_sh):
        cid = lax.axis_index("c")
        sid = lax.axis_index("s")
        wid = sid * NC + cid
        pltpu.sync_copy(idx_hbm.at[pl.ds(wid * ncpw, ncpw)], idx_v)
        pltpu.sync_copy(
            z_hbm.at[pl.ds(sid * rpt, rpt)], acc_sh.at[pl.ds(sid * rpt, rpt)]
        )
        plsc.subcore_barrier()

        def chunk(i, _):
            pltpu.sync_copy(m_hbm.at[pl.ds((wid * ncpw + i) * CH, CH)], rows_v)
            pltpu.sync_copy(rows_v, acc_sh.at[idx_v.at[i]], add=True)
            return _

        lax.fori_loop(0, ncpw, chunk, 0)
        plsc.subcore_barrier()
        pltpu.sync_copy(
            acc_sh.at[pl.ds(sid * rpt, rpt)], out_hbm.at[cid, pl.ds(sid * rpt, rpt)]
        )

    f = pl.kernel(
        body,
        out_type=jax.ShapeDtypeStruct((NC, nr, MD), _f32),
        mesh=_mesh(),
        scratch_types=[
            pltpu.VMEM((ncpw, CH), _i32),
            pltpu.VMEM((CH, MD), _f32),
            pltpu.VMEM_SHARED((nr, MD), _f32),
        ],
        compiler_params=pltpu.CompilerParams(use_tc_tiling_on_sc=False),
    )
    return f(m, idx2d, zeros)


def _silu(x):
    return x * jax.nn.sigmoid(x)


def _dot(a, b):
    return jnp.dot(a, b, preferred_element_type=_f32)


def _dot_exact(a, b):
    return lax.dot_general(
        a, b, (((1,), (0,)), ((), ())), precision="highest",
        preferred_element_type=_f32,
    )


def _fold_sum(x):
    w = x.shape[1]
    while w > 1:
        x = x[:, : w // 2] + x[:, w // 2 :]
        w //= 2
    return x


def _lnorm(x, g, b):
    n = x.shape[-1]
    mu = _fold_sum(x) / float(n)
    xc = x - mu
    var = _fold_sum(xc * xc) / float(n)
    return xc / jnp.sqrt(var + 1e-5) * g + b


def _full(shape):
    nd = len(shape)
    return pl.BlockSpec(shape, lambda i: (0,) * nd)


def _pre_body(ai_ref, ri_ref, pp_ref, ae, re, pw, pb, p1, b1, p2, b2, o_ref, ob_ref):
    ai = ai_ref[0, 0, :].reshape(BE, 1)
    ri = ri_ref[0, 0, :].reshape(BE, 1)
    oh_a = (ai == lax.broadcasted_iota(_i32, (1, 32), 1)).astype(_f32)
    oh_r = (ri == lax.broadcasted_iota(_i32, (1, 256), 1)).astype(_f32)
    fa = _dot_exact(oh_a, ae[...])
    fr = _dot_exact(oh_r, re[...])
    fp = _dot(pp_ref[...], pw[...]) + pb[...]
    cat = jnp.concatenate([fa, fr, fp], axis=-1)
    t = _silu(_dot(cat, p1[...]) + b1[...])
    f = _dot(t, p2[...]) + b2[...]
    o_ref[...] = f
    ob_ref[...] = f.astype(jnp.bfloat16)


def _ea_body(rd_ref, o_ref):
    d = rd_ref[...]
    inv = jnp.exp2(-lax.broadcasted_iota(_i32, (1, 16), 1).astype(_f32))
    xs = d * inv
    o_ref[...] = jnp.concatenate(
        [jnp.sin(xs), jnp.cos(xs), d, jnp.zeros((BE, 31), _f32)], axis=-1
    )


def _edge_body(xj_ref, xi_ref, ea_ref, w1ab, w1c, b1, w2, b2, g1, bb1, o_ref):
    cat = jnp.concatenate([xi_ref[...], xj_ref[...]], axis=-1)
    h = (_dot(cat, w1ab[...]) + _dot(ea_ref[...], w1c[...])) + b1[...]
    m = _silu(_dot(_silu(h), w2[...]) + b2[...])
    o_ref[...] = _lnorm(m, g1[...], bb1[...])


def _node_body(f_ref, mga_ref, mgb_ref, n1, nb1, n2, nb2, ge2, be2, gn1, bn1, gn2, bn2, o_ref, ob_ref):
    f = f_ref[...]
    msum = (mga_ref[0] + mga_ref[1]) + (mgb_ref[0] + mgb_ref[1])
    mi = _lnorm(msum, ge2[...], be2[...])
    hid = _lnorm(f, gn1[...], bn1[...])
    cat = jnp.concatenate([hid, mi], axis=-1)
    t = _silu(_dot(cat, n1[...]) + nb1[...])
    ho = _dot(t, n2[...]) + nb2[...]
    fo = f + _lnorm(ho, gn2[...], bn2[...])
    o_ref[...] = fo
    ob_ref[...] = fo.astype(jnp.bfloat16)


def _post_body(f1_ref, f2_ref, f3_ref, q1ab, q1c, c1, q2, c2, q3, c3, o_ref):
    cat = jnp.concatenate([f1_ref[...], f2_ref[...]], axis=-1)
    h = _silu((_dot(cat, q1ab[...]) + _dot(f3_ref[...], q1c[...])) + c1[...])
    h = _silu(_dot(h, q2[...]) + c2[...])
    o_ref[...] = _silu(_dot(h, q3[...]) + c3[...])


def _row(v):
    return v.reshape(1, -1)


def _pad2(w, r, c):
    return jnp.pad(w, ((0, r - w.shape[0]), (0, c - w.shape[1])))


def kernel(pos, props, atom_idx, residue_idx, edge_index, params):
    src = edge_index[0].astype(_i32)
    dst = edge_index[1].astype(_i32)

    eph = EP // 2
    eh = E // 2
    padzh = jnp.zeros((eph - eh,), _i32)
    dumh = jnp.full((eph - eh,), N, _i32)
    idx_h = [
        jnp.concatenate([src[h * eh : (h + 1) * eh], padzh,
                         dst[h * eh : (h + 1) * eh], padzh]).reshape(2 * eph // CH, CH)
        for h in range(2)
    ]
    dst_h = [
        jnp.concatenate([dst[h * eh : (h + 1) * eh], dumh]).reshape(eph // CH, CH)
        for h in range(2)
    ]

    pos_pad = jnp.pad(pos, ((0, NPAD - N), (0, 13)))
    props_pad = jnp.pad(props, ((0, NPAD - N), (0, 6)))
    ai3 = jnp.pad(atom_idx.astype(_i32), (0, NPAD - N)).reshape(NPAD // BE, 1, BE)
    ri3 = jnp.pad(residue_idx.astype(_i32), (0, NPAD - N)).reshape(NPAD // BE, 1, BE)
    zeros_nr = jnp.zeros((NPAD, MD), _f32)

    p = params
    grid_n = (NPAD // BE,)
    grid_e = (NBLK,)

    pre_ws = [
        _pad2(p["atom_emb"], 32, 64),
        _pad2(p["residue_emb"], 256, 64),
        _pad2(p["prop_lin"]["w"], 8, 32),
        _row(p["prop_lin"]["b"]),
        p["pre1"]["w"],
        _row(p["pre1"]["b"]),
        p["pre2"]["w"],
        _row(p["pre2"]["b"]),
    ]
    feats, featsb = pl.pallas_call(
        _pre_body,
        grid=grid_n,
        in_specs=[
            pl.BlockSpec((1, 1, BE), lambda i: (i, 0, 0)),
            pl.BlockSpec((1, 1, BE), lambda i: (i, 0, 0)),
            pl.BlockSpec((BE, 8), lambda i: (i, 0)),
        ]
        + [_full(w.shape) for w in pre_ws],
        out_specs=[pl.BlockSpec((BE, KD), lambda i: (i, 0))] * 2,
        out_shape=[
            jax.ShapeDtypeStruct((NPAD, KD), _f32),
            jax.ShapeDtypeStruct((NPAD, KD), jnp.bfloat16),
        ],
    )(ai3, ri3, props_pad, *pre_ws)

    grid_h = (eph // BE,)
    ea_h = []
    posg_h = [_sc_gather(pos_pad, idx_h[h], ncpw=2 * eph // NW // CH, d=16)
              for h in range(2)]
    for h in range(2):
        rel3 = posg_h[h][:eph, :3] - posg_h[h][eph:, :3]
        rd = jnp.sum(rel3**2, axis=-1, keepdims=True)
        ea_h.append(pl.pallas_call(
            _ea_body,
            grid=grid_h,
            in_specs=[pl.BlockSpec((BE, 1), lambda i: (i, 0))],
            out_specs=pl.BlockSpec((BE, 64), lambda i: (i, 0)),
            out_shape=jax.ShapeDtypeStruct((eph, 64), _f32),
        )(rd))

    feat_list = []
    nblk_h = eph // BE
    for kp in p["kernels"]:
        gaths = [_sc_gather(featsb, idx_h[h], ncpw=2 * eph // NW // CH, d=KD)
                 for h in range(2)]
        edge_ws = [
            _pad2(kp["e1"]["w"][:256], 256, D1).astype(jnp.bfloat16),
            _pad2(kp["e1"]["w"][256:], 64, D1),
            _row(jnp.pad(kp["e1"]["b"], (0, D1 - 2 * 289))),
            _pad2(kp["e2"]["w"], D1, MD),
            _row(kp["e2"]["b"]),
            _row(kp["ln_e1_g"]),
            _row(kp["ln_e1_b"]),
        ]
        maggs = []
        for h in range(2):
            m = pl.pallas_call(
                _edge_body,
                grid=grid_h,
                in_specs=[
                    pl.BlockSpec((BE, KD), lambda i: (i, 0)),
                    pl.BlockSpec((BE, KD), lambda i, _n=nblk_h: (i + _n, 0)),
                    pl.BlockSpec((BE, 64), lambda i: (i, 0)),
                ]
                + [_full(w.shape) for w in edge_ws],
                out_specs=pl.BlockSpec((BE, MD), lambda i: (i, 0)),
                out_shape=jax.ShapeDtypeStruct((eph, MD), _f32),
            )(gaths[h], gaths[h], ea_h[h], *edge_ws)
            maggs.append(
                _sc_scatter_add(m, dst_h[h], zeros_nr, ncpw=eph // NW // CH, nr=NPAD)
            )

        node_ws = [
            kp["n1"]["w"],
            _row(kp["n1"]["b"]),
            kp["n2"]["w"],
            _row(kp["n2"]["b"]),
            _row(kp["ln_e2_g"]),
            _row(kp["ln_e2_b"]),
            _row(kp["ln_n1_g"]),
            _row(kp["ln_n1_b"]),
            _row(kp["ln_n2_g"]),
            _row(kp["ln_n2_b"]),
        ]
        feats, featsb = pl.pallas_call(
            _node_body,
            grid=grid_n,
            in_specs=[
                pl.BlockSpec((BE, KD), lambda i: (i, 0)),
                pl.BlockSpec((NC, BE, MD), lambda i: (0, i, 0)),
                pl.BlockSpec((NC, BE, MD), lambda i: (0, i, 0)),
            ]
            + [_full(w.shape) for w in node_ws],
            out_specs=[pl.BlockSpec((BE, KD), lambda i: (i, 0))] * 2,
            out_shape=[
                jax.ShapeDtypeStruct((NPAD, KD), _f32),
                jax.ShapeDtypeStruct((NPAD, KD), jnp.bfloat16),
            ],
        )(feats, maggs[0], maggs[1], *node_ws)
        feat_list.append(feats)

    post_ws = [
        p["post1"]["w"][: 2 * KD],
        p["post1"]["w"][2 * KD :],
        _row(p["post1"]["b"]),
        p["post2"]["w"],
        _row(p["post2"]["b"]),
        p["post3"]["w"],
        _row(p["post3"]["b"]),
    ]
    h = pl.pallas_call(
        _post_body,
        grid=grid_n,
        in_specs=[pl.BlockSpec((BE, KD), lambda i: (i, 0))] * 3
        + [_full(w.shape) for w in post_ws],
        out_specs=pl.BlockSpec((BE, KD), lambda i: (i, 0)),
        out_shape=jax.ShapeDtypeStruct((NPAD, KD), _f32),
    )(*feat_list, *post_ws)
    return h[:N]


# speedup vs baseline: 1.0919x; 1.0919x over previous
import functools

import jax
import jax.numpy as jnp
from jax import lax
from jax.experimental import pallas as pl
from jax.experimental.pallas import tpu as pltpu
from jax.experimental.pallas import tpu_sc as plsc

N = 10000
E = 320000
KD = 128
MD = 16
NPAD = 10240
EP = 327680
BG = 2 * EP
BE = 512
NBLK = EP // BE
D1 = 640
NC = 2
NS = 16
NW = NC * NS
CH = 128

_f32 = jnp.float32
_i32 = jnp.int32


def _mesh():
    return plsc.VectorSubcoreMesh(
        core_axis_name="c", subcore_axis_name="s", num_cores=NC, num_subcores=NS
    )


@functools.partial(jax.jit, static_argnames=("ncpw", "d"))
def _sc_gather(table, idx2d, *, ncpw, d):
    b = idx2d.shape[0] * CH
    kfire = 5

    def body(table_hbm, idx_hbm, out_hbm, idx_v, rows_v, sems):
        wid = lax.axis_index("s") * NC + lax.axis_index("c")
        pltpu.sync_copy(idx_hbm.at[pl.ds(wid * ncpw, ncpw)], idx_v)

        def chunk(i, _):
            c0 = wid * ncpw + i * kfire
            descs = [
                pltpu.async_copy(
                    table_hbm.at[idx_v.at[i * kfire + k]], rows_v.at[k], sems.at[k]
                )
                for k in range(kfire)
            ]
            for k in range(kfire):
                descs[k].wait()
                pltpu.sync_copy(rows_v.at[k], out_hbm.at[pl.ds((c0 + k) * CH, CH)])
            return _

        lax.fori_loop(0, ncpw // kfire, chunk, 0)

    f = pl.kernel(
        body,
        out_type=jax.ShapeDtypeStruct((b, d), table.dtype),
        mesh=_mesh(),
        scratch_types=[
            pltpu.VMEM((ncpw, CH), _i32),
            pltpu.VMEM((kfire, CH, d), table.dtype),
            pltpu.SemaphoreType.DMA((kfire,)),
        ],
        compiler_params=pltpu.CompilerParams(use_tc_tiling_on_sc=False),
    )
    return f(table, idx2d)


@functools.partial(jax.jit, static_argnames=("ncpw", "nr"))
def _sc_scatter_add(m, idx2d, zeros, *, ncpw, nr):
    rpt = nr // NS

    def body(m_hbm, idx_hbm, z_hbm, out_hbm, idx_v, rows_v, acc_sh):
        cid = lax.axis_index("c")
        sid = lax.axis_index("s")
        wid = sid * NC + cid
        pltpu.sync_copy(idx_hbm.at[pl.ds(wid * ncpw, ncpw)], idx_v)
        pltpu.sync_copy(
            z_hbm.at[pl.ds(sid * rpt, rpt)], acc_sh.at[pl.ds(sid * rpt, rpt)]
        )
        plsc.subcore_barrier()

        def chunk(i, _):
            pltpu.sync_copy(m_hbm.at[pl.ds((wid * ncpw + i) * CH, CH)], rows_v)
            pltpu.sync_copy(rows_v, acc_sh.at[idx_v.at[i]], add=True)
            return _

        lax.fori_loop(0, ncpw, chunk, 0)
        plsc.subcore_barrier()
        pltpu.sync_copy(
            acc_sh.at[pl.ds(sid * rpt, rpt)], out_hbm.at[cid, pl.ds(sid * rpt, rpt)]
        )

    f = pl.kernel(
        body,
        out_type=jax.ShapeDtypeStruct((NC, nr, MD), _f32),
        mesh=_mesh(),
        scratch_types=[
            pltpu.VMEM((ncpw, CH), _i32),
            pltpu.VMEM((CH, MD), _f32),
            pltpu.VMEM_SHARED((nr, MD), _f32),
        ],
        compiler_params=pltpu.CompilerParams(use_tc_tiling_on_sc=False),
    )
    return f(m, idx2d, zeros)


def _silu(x):
    return x * jax.nn.sigmoid(x)


def _dot(a, b):
    return jnp.dot(a, b, preferred_element_type=_f32)


def _dot_exact(a, b):
    return lax.dot_general(
        a, b, (((1,), (0,)), ((), ())), precision="highest",
        preferred_element_type=_f32,
    )


def _fold_sum(x):
    w = x.shape[1]
    while w > 1:
        x = x[:, : w // 2] + x[:, w // 2 :]
        w //= 2
    return x


def _lnorm(x, g, b):
    n = x.shape[-1]
    mu = _fold_sum(x) / float(n)
    xc = x - mu
    var = _fold_sum(xc * xc) / float(n)
    return xc / jnp.sqrt(var + 1e-5) * g + b


def _full(shape):
    nd = len(shape)
    return pl.BlockSpec(shape, lambda i: (0,) * nd)


def _pre_body(ai_ref, ri_ref, pp_ref, ae, re, pw, pb, p1, b1, p2, b2, o_ref, ob_ref):
    ai = ai_ref[0, 0, :].reshape(BE, 1)
    ri = ri_ref[0, 0, :].reshape(BE, 1)
    oh_a = (ai == lax.broadcasted_iota(_i32, (1, 32), 1)).astype(_f32)
    oh_r = (ri == lax.broadcasted_iota(_i32, (1, 256), 1)).astype(_f32)
    fa = _dot_exact(oh_a, ae[...])
    fr = _dot_exact(oh_r, re[...])
    fp = _dot(pp_ref[...], pw[...]) + pb[...]
    cat = jnp.concatenate([fa, fr, fp], axis=-1)
    t = _silu(_dot(cat, p1[...]) + b1[...])
    f = _dot(t, p2[...]) + b2[...]
    o_ref[...] = f
    ob_ref[...] = f.astype(jnp.bfloat16)


def _ea_body(rd_ref, o_ref):
    d = rd_ref[...]
    inv = jnp.exp2(-lax.broadcasted_iota(_i32, (1, 16), 1).astype(_f32))
    xs = d * inv
    o_ref[...] = jnp.concatenate(
        [jnp.sin(xs), jnp.cos(xs), d, jnp.zeros((BE, 31), _f32)], axis=-1
    )


def _edge_body(xj_ref, xi_ref, ea_ref, w1ab, w1c, b1, w2, b2, g1, bb1, o_ref):
    cat = jnp.concatenate([xi_ref[...], xj_ref[...]], axis=-1)
    h = (_dot(cat, w1ab[...]) + _dot(ea_ref[...], w1c[...])) + b1[...]
    m = _silu(_dot(_silu(h), w2[...]) + b2[...])
    o_ref[...] = _lnorm(m, g1[...], bb1[...])


def _node_body(f_ref, mga_ref, mgb_ref, n1, nb1, n2, nb2, ge2, be2, gn1, bn1, gn2, bn2, o_ref, ob_ref):
    f = f_ref[...]
    msum = (mga_ref[0] + mga_ref[1]) + (mgb_ref[0] + mgb_ref[1])
    mi = _lnorm(msum, ge2[...], be2[...])
    hid = _lnorm(f, gn1[...], bn1[...])
    cat = jnp.concatenate([hid, mi], axis=-1)
    t = _silu(_dot(cat, n1[...]) + nb1[...])
    ho = _dot(t, n2[...]) + nb2[...]
    fo = f + _lnorm(ho, gn2[...], bn2[...])
    o_ref[...] = fo
    ob_ref[...] = fo.astype(jnp.bfloat16)


def _post_body(f1_ref, f2_ref, f3_ref, q1ab, q1c, c1, q2, c2, q3, c3, o_ref):
    cat = jnp.concatenate([f1_ref[...], f2_ref[...]], axis=-1)
    h = _silu((_dot(cat, q1ab[...]) + _dot(f3_ref[...], q1c[...])) + c1[...])
    h = _silu(_dot(h, q2[...]) + c2[...])
    o_ref[...] = _silu(_dot(h, q3[...]) + c3[...])


def _row(v):
    return v.reshape(1, -1)


def _pad2(w, r, c):
    return jnp.pad(w, ((0, r - w.shape[0]), (0, c - w.shape[1])))


def kernel(pos, props, atom_idx, residue_idx, edge_index, params):
    src = edge_index[0].astype(_i32)
    dst = edge_index[1].astype(_i32)

    eph = EP // 2
    eh = E // 2
    padzh = jnp.zeros((eph - eh,), _i32)
    dumh = jnp.full((eph - eh,), N, _i32)
    idx_h = [
        jnp.concatenate([src[h * eh : (h + 1) * eh], padzh,
                         dst[h * eh : (h + 1) * eh], padzh]).reshape(2 * eph // CH, CH)
        for h in range(2)
    ]
    dst_h = [
        jnp.concatenate([dst[h * eh : (h + 1) * eh], dumh]).reshape(eph // CH, CH)
        for h in range(2)
    ]

    pos_pad = jnp.pad(pos, ((0, NPAD - N), (0, 13)))
    props_pad = jnp.pad(props, ((0, NPAD - N), (0, 6)))
    ai3 = jnp.pad(atom_idx.astype(_i32), (0, NPAD - N)).reshape(NPAD // BE, 1, BE)
    ri3 = jnp.pad(residue_idx.astype(_i32), (0, NPAD - N)).reshape(NPAD // BE, 1, BE)
    zeros_nr = jnp.zeros((NPAD, MD), _f32)

    p = params
    grid_n = (NPAD // BE,)
    grid_e = (NBLK,)

    pre_ws = [
        _pad2(p["atom_emb"], 32, 64),
        _pad2(p["residue_emb"], 256, 64),
        _pad2(p["prop_lin"]["w"], 8, 32),
        _row(p["prop_lin"]["b"]),
        p["pre1"]["w"],
        _row(p["pre1"]["b"]),
        p["pre2"]["w"],
        _row(p["pre2"]["b"]),
    ]
    feats, featsb = pl.pallas_call(
        _pre_body,
        grid=grid_n,
        in_specs=[
            pl.BlockSpec((1, 1, BE), lambda i: (i, 0, 0)),
            pl.BlockSpec((1, 1, BE), lambda i: (i, 0, 0)),
            pl.BlockSpec((BE, 8), lambda i: (i, 0)),
        ]
        + [_full(w.shape) for w in pre_ws],
        out_specs=[pl.BlockSpec((BE, KD), lambda i: (i, 0))] * 2,
        out_shape=[
            jax.ShapeDtypeStruct((NPAD, KD), _f32),
            jax.ShapeDtypeStruct((NPAD, KD), jnp.bfloat16),
        ],
    )(ai3, ri3, props_pad, *pre_ws)

    grid_h = (eph // BE,)
    ea_h = []
    posg_h = [_sc_gather(pos_pad, idx_h[h], ncpw=2 * eph // NW // CH, d=16)
              for h in range(2)]
    for h in range(2):
        rel3 = posg_h[h][:eph, :3] - posg_h[h][eph:, :3]
        rd = jnp.sum(rel3**2, axis=-1, keepdims=True)
        ea_h.append(pl.pallas_call(
            _ea_body,
            grid=grid_h,
            in_specs=[pl.BlockSpec((BE, 1), lambda i: (i, 0))],
            out_specs=pl.BlockSpec((BE, 64), lambda i: (i, 0)),
            out_shape=jax.ShapeDtypeStruct((eph, 64), _f32),
        )(rd))

    feat_list = []
    nblk_h = eph // BE
    for kp in p["kernels"]:
        gaths = [_sc_gather(feats, idx_h[h], ncpw=2 * eph // NW // CH, d=KD)
                 for h in range(2)]
        edge_ws = [
            _pad2(kp["e1"]["w"][:256], 256, D1),
            _pad2(kp["e1"]["w"][256:], 64, D1),
            _row(jnp.pad(kp["e1"]["b"], (0, D1 - 2 * 289))),
            _pad2(kp["e2"]["w"], D1, MD),
            _row(kp["e2"]["b"]),
            _row(kp["ln_e1_g"]),
            _row(kp["ln_e1_b"]),
        ]
        maggs = []
        for h in range(2):
            m = pl.pallas_call(
                _edge_body,
                grid=grid_h,
                in_specs=[
                    pl.BlockSpec((BE, KD), lambda i: (i, 0)),
                    pl.BlockSpec((BE, KD), lambda i, _n=nblk_h: (i + _n, 0)),
                    pl.BlockSpec((BE, 64), lambda i: (i, 0)),
                ]
                + [_full(w.shape) for w in edge_ws],
                out_specs=pl.BlockSpec((BE, MD), lambda i: (i, 0)),
                out_shape=jax.ShapeDtypeStruct((eph, MD), _f32),
            )(gaths[h], gaths[h], ea_h[h], *edge_ws)
            maggs.append(
                _sc_scatter_add(m, dst_h[h], zeros_nr, ncpw=eph // NW // CH, nr=NPAD)
            )

        node_ws = [
            kp["n1"]["w"],
            _row(kp["n1"]["b"]),
            kp["n2"]["w"],
            _row(kp["n2"]["b"]),
            _row(kp["ln_e2_g"]),
            _row(kp["ln_e2_b"]),
            _row(kp["ln_n1_g"]),
            _row(kp["ln_n1_b"]),
            _row(kp["ln_n2_g"]),
            _row(kp["ln_n2_b"]),
        ]
        feats, featsb = pl.pallas_call(
            _node_body,
            grid=grid_n,
            in_specs=[
                pl.BlockSpec((BE, KD), lambda i: (i, 0)),
                pl.BlockSpec((NC, BE, MD), lambda i: (0, i, 0)),
                pl.BlockSpec((NC, BE, MD), lambda i: (0, i, 0)),
            ]
            + [_full(w.shape) for w in node_ws],
            out_specs=[pl.BlockSpec((BE, KD), lambda i: (i, 0))] * 2,
            out_shape=[
                jax.ShapeDtypeStruct((NPAD, KD), _f32),
                jax.ShapeDtypeStruct((NPAD, KD), jnp.bfloat16),
            ],
        )(feats, maggs[0], maggs[1], *node_ws)
        feat_list.append(feats)

    post_ws = [
        p["post1"]["w"][: 2 * KD],
        p["post1"]["w"][2 * KD :],
        _row(p["post1"]["b"]),
        p["post2"]["w"],
        _row(p["post2"]["b"]),
        p["post3"]["w"],
        _row(p["post3"]["b"]),
    ]
    h = pl.pallas_call(
        _post_body,
        grid=grid_n,
        in_specs=[pl.BlockSpec((BE, KD), lambda i: (i, 0))] * 3
        + [_full(w.shape) for w in post_ws],
        out_specs=pl.BlockSpec((BE, KD), lambda i: (i, 0)),
        out_shape=jax.ShapeDtypeStruct((NPAD, KD), _f32),
    )(*feat_list, *post_ws)
    return h[:N]
